# baseline (speedup 1.0000x reference)
import jax
import jax.numpy as jnp
from jax import lax
from jax.experimental import pallas as pl
from jax.experimental.pallas import tpu as pltpu

_ROWS_PER_BLK = 8


def _nca_body(y_ref, idx_ref, x_ref, labels_ref, out_ref):
    y = y_ref[0, 0, :]
    idx = idx_ref[0, 0, :]
    x = x_ref[...]
    labels = labels_ref[0, :]
    same = y[:, None] == labels[None, :]
    col = lax.broadcasted_iota(jnp.int32, x.shape, 1)
    keep = same & (col != idx[:, None])
    out_ref[0, 0, :] = jnp.sum(jnp.where(keep, jnp.exp(x), 0.0), axis=1)


def kernel(x, features, indexes, labels):
    del features
    B, N = x.shape
    nblk = B // _ROWS_PER_BLK
    idx32 = indexes.astype(jnp.int32)
    lab32 = labels.astype(jnp.int32)
    y = jnp.take(lab32, idx32, axis=0)
    y3 = y.reshape(nblk, 1, _ROWS_PER_BLK)
    idx3 = idx32.reshape(nblk, 1, _ROWS_PER_BLK)
    lab2 = lab32.reshape(1, N)

    out = pl.pallas_call(
        _nca_body,
        grid=(nblk,),
        in_specs=[
            pl.BlockSpec((1, 1, _ROWS_PER_BLK), lambda i: (i, 0, 0)),
            pl.BlockSpec((1, 1, _ROWS_PER_BLK), lambda i: (i, 0, 0)),
            pl.BlockSpec((_ROWS_PER_BLK, N), lambda i: (i, 0)),
            pl.BlockSpec((1, N), lambda i: (0, 0)),
        ],
        out_specs=pl.BlockSpec((1, 1, _ROWS_PER_BLK), lambda i: (i, 0, 0)),
        out_shape=jax.ShapeDtypeStruct((nblk, 1, _ROWS_PER_BLK), jnp.float32),
    )(y3, idx3, x, lab2)
    return out.reshape(B)
